# baseline (speedup 1.0000x reference)
import functools

import jax
import jax.numpy as jnp
from jax import lax
from jax.experimental import pallas as pl
from jax.experimental.pallas import tpu as pltpu
from jax.experimental.pallas import tpu_sc as plsc

N_ATOMS = 40000
N_BLOCKS = 10000
N_GRAPHS = 256
E = 320000
H_DIM = 128
T_DIM = 144
XD = 16


def _silu(x):
    return x * jax.nn.sigmoid(x)


def _mask3():
    return (jax.lax.broadcasted_iota(jnp.int32, (1, XD), 1) < 3).astype(jnp.float32)


def _onehot3():
    return (jax.lax.broadcasted_iota(jnp.int32, (1, XD), 1) == 3).astype(jnp.float32)


def _k2_body(s_ref, emb_w_ref, emb_b_ref, wa_ref, wb_ref, b1_ref,
             hb_ref, xp_ref, h_ref, fa_ref, fb_ref):
    s = jnp.sum(s_ref[...], axis=0)
    cnt = jnp.maximum(s[:, 131:132], 1.0)
    hb = s[:, :H_DIM] / cnt
    xp = (s[:, H_DIM:] / cnt) * _mask3()
    hb_ref[...] = hb
    xp_ref[...] = xp
    h = jnp.dot(hb, emb_w_ref[...], preferred_element_type=jnp.float32) + emb_b_ref[...]
    h_ref[...] = h
    a = jnp.dot(h, wa_ref[...], preferred_element_type=jnp.float32) + b1_ref[...]
    b = jnp.dot(h, wb_ref[...], preferred_element_type=jnp.float32)
    fa_ref[...] = a.astype(jnp.bfloat16)
    fb_ref[...] = b.astype(jnp.bfloat16)


def _k2(s, emb_w, emb_b, wa, wb, b1):
    B = 1000
    grid = (N_BLOCKS // B,)
    w_spec = pl.BlockSpec((H_DIM, H_DIM), lambda i: (0, 0))
    bias_spec = pl.BlockSpec((1, H_DIM), lambda i: (0, 0))
    return pl.pallas_call(
        _k2_body,
        grid=grid,
        in_specs=[
            pl.BlockSpec((s.shape[0], B, T_DIM), lambda i: (0, i, 0)),
            w_spec, bias_spec, w_spec, w_spec, bias_spec,
        ],
        out_specs=[
            pl.BlockSpec((B, H_DIM), lambda i: (i, 0)),
            pl.BlockSpec((B, XD), lambda i: (i, 0)),
            pl.BlockSpec((B, H_DIM), lambda i: (i, 0)),
            pl.BlockSpec((B, H_DIM), lambda i: (i, 0)),
            pl.BlockSpec((B, H_DIM), lambda i: (i, 0)),
        ],
        out_shape=[
            jax.ShapeDtypeStruct((N_BLOCKS, H_DIM), jnp.float32),
            jax.ShapeDtypeStruct((N_BLOCKS, XD), jnp.float32),
            jax.ShapeDtypeStruct((N_BLOCKS, H_DIM), jnp.float32),
            jax.ShapeDtypeStruct((N_BLOCKS, H_DIM), jnp.bfloat16),
            jax.ShapeDtypeStruct((N_BLOCKS, H_DIM), jnp.bfloat16),
        ],
    )(s, emb_w, emb_b.reshape(1, -1), wa, wb, b1.reshape(1, -1))


def _k4_body(raf_ref, rbf_ref, xr_ref, xc_ref, ea_ref, we_ref, wr_ref,
             w2_ref, b2_ref, c1_ref, cb1_ref, c2_ref, out_ref):
    d = xr_ref[...] - xc_ref[...]
    radial = jnp.sum(d * d, axis=1, keepdims=True)
    h1 = raf_ref[...].astype(jnp.float32) + rbf_ref[...].astype(jnp.float32)
    m1 = h1 + radial * wr_ref[...] + jnp.dot(
        ea_ref[...], we_ref[...], preferred_element_type=jnp.float32)
    m1 = _silu(m1).astype(jnp.bfloat16)
    m = _silu(jnp.dot(m1, w2_ref[...], preferred_element_type=jnp.float32) + b2_ref[...])
    p = _silu(jnp.dot(m.astype(jnp.bfloat16), c1_ref[...],
                      preferred_element_type=jnp.float32) + cb1_ref[...])
    phi = jnp.dot(p, c2_ref[...], preferred_element_type=jnp.float32)
    tp = d * phi + _onehot3()
    out_ref[...] = jnp.concatenate([m, tp], axis=1)


def _k4(raf, rbf, xr, xc, ea, we, wr, w2b, b2, c1b, cb1, c2):
    B = 4000
    grid = (E // B,)
    return pl.pallas_call(
        _k4_body,
        grid=grid,
        in_specs=[
            pl.BlockSpec((B, H_DIM), lambda i: (i, 0)),
            pl.BlockSpec((B, H_DIM), lambda i: (i, 0)),
            pl.BlockSpec((B, XD), lambda i: (i, 0)),
            pl.BlockSpec((B, XD), lambda i: (i, 0)),
            pl.BlockSpec((B, 16), lambda i: (i, 0)),
            pl.BlockSpec((16, H_DIM), lambda i: (0, 0)),
            pl.BlockSpec((1, H_DIM), lambda i: (0, 0)),
            pl.BlockSpec((H_DIM, H_DIM), lambda i: (0, 0)),
            pl.BlockSpec((1, H_DIM), lambda i: (0, 0)),
            pl.BlockSpec((H_DIM, H_DIM), lambda i: (0, 0)),
            pl.BlockSpec((1, H_DIM), lambda i: (0, 0)),
            pl.BlockSpec((H_DIM, 1), lambda i: (0, 0)),
        ],
        out_specs=pl.BlockSpec((B, T_DIM), lambda i: (i, 0)),
        out_shape=jax.ShapeDtypeStruct((E, T_DIM), jnp.float32),
    )(raf, rbf, xr, xc, ea, we, wr.reshape(1, -1), w2b, b2.reshape(1, -1),
      c1b, cb1.reshape(1, -1), c2)


def _k6_body(last, s_ref, h_ref, xp_ref, wn1a_ref, wn1b_ref, nb1_ref,
             wn2_ref, nb2_ref, nwa_ref, nwb_ref, nb1n_ref,
             h_out, xp_out, fa_ref, fb_ref, br_ref):
    s = jnp.sum(s_ref[...], axis=0)
    cnt = jnp.maximum(s[:, 131:132], 1.0)
    aggm = s[:, :H_DIM]
    xp = xp_ref[...] + (s[:, H_DIM:] * _mask3()) / cnt
    h = h_ref[...]
    hid = _silu(jnp.dot(h, wn1a_ref[...], preferred_element_type=jnp.float32)
                + jnp.dot(aggm, wn1b_ref[...], preferred_element_type=jnp.float32)
                + nb1_ref[...])
    h = h + jnp.dot(hid, wn2_ref[...], preferred_element_type=jnp.float32) + nb2_ref[...]
    h_out[...] = h
    xp_out[...] = xp
    a = jnp.dot(h, nwa_ref[...], preferred_element_type=jnp.float32) + nb1n_ref[...]
    if last:
        nrm = jax.lax.rsqrt(jnp.maximum(jnp.sum(a * a, axis=1, keepdims=True), 1e-24))
        br_ref[...] = a * nrm
        fa_ref[...] = jnp.zeros(fa_ref.shape, fa_ref.dtype)
        fb_ref[...] = jnp.zeros(fb_ref.shape, fb_ref.dtype)
    else:
        b = jnp.dot(h, nwb_ref[...], preferred_element_type=jnp.float32)
        fa_ref[...] = a.astype(jnp.bfloat16)
        fb_ref[...] = b.astype(jnp.bfloat16)
        br_ref[...] = jnp.zeros(br_ref.shape, br_ref.dtype)


def _k6(s, h, xp, wn1a, wn1b, nb1, wn2, nb2, nwa, nwb, nb1n, last):
    B = 1000
    grid = (N_BLOCKS // B,)
    w_spec = pl.BlockSpec((H_DIM, H_DIM), lambda i: (0, 0))
    bias_spec = pl.BlockSpec((1, H_DIM), lambda i: (0, 0))
    return pl.pallas_call(
        functools.partial(_k6_body, last),
        grid=grid,
        in_specs=[
            pl.BlockSpec((s.shape[0], B, T_DIM), lambda i: (0, i, 0)),
            pl.BlockSpec((B, H_DIM), lambda i: (i, 0)),
            pl.BlockSpec((B, XD), lambda i: (i, 0)),
            w_spec, w_spec, bias_spec, w_spec, bias_spec,
            w_spec, w_spec, bias_spec,
        ],
        out_specs=[
            pl.BlockSpec((B, H_DIM), lambda i: (i, 0)),
            pl.BlockSpec((B, XD), lambda i: (i, 0)),
            pl.BlockSpec((B, H_DIM), lambda i: (i, 0)),
            pl.BlockSpec((B, H_DIM), lambda i: (i, 0)),
            pl.BlockSpec((B, H_DIM), lambda i: (i, 0)),
        ],
        out_shape=[
            jax.ShapeDtypeStruct((N_BLOCKS, H_DIM), jnp.float32),
            jax.ShapeDtypeStruct((N_BLOCKS, XD), jnp.float32),
            jax.ShapeDtypeStruct((N_BLOCKS, H_DIM), jnp.bfloat16),
            jax.ShapeDtypeStruct((N_BLOCKS, H_DIM), jnp.bfloat16),
            jax.ShapeDtypeStruct((N_BLOCKS, H_DIM), jnp.float32),
        ],
    )(s, h, xp, wn1a, wn1b, nb1.reshape(1, -1), wn2, nb2.reshape(1, -1),
      nwa, nwb, nb1n.reshape(1, -1))


def _k7_body(br_ref, bid_ref, acc_ref, out_ref):
    i = pl.program_id(0)
    bid = bid_ref[0]
    onehot = (jax.lax.broadcasted_iota(jnp.int32, (N_GRAPHS, bid.shape[1]), 0)
              == bid).astype(jnp.float32)
    part = jnp.dot(onehot, br_ref[...], preferred_element_type=jnp.float32)

    @pl.when(i == 0)
    def _init():
        acc_ref[...] = part

    @pl.when(i > 0)
    def _acc():
        acc_ref[...] = acc_ref[...] + part

    @pl.when(i == pl.num_programs(0) - 1)
    def _fin():
        g = acc_ref[...]
        nrm = jax.lax.rsqrt(jnp.maximum(jnp.sum(g * g, axis=1, keepdims=True), 1e-24))
        out_ref[...] = g * nrm


def _k7(br, bid2d):
    B = 2000
    grid = (N_BLOCKS // B,)
    bid3 = bid2d.reshape(N_BLOCKS // B, 1, B)
    return pl.pallas_call(
        _k7_body,
        grid=grid,
        in_specs=[
            pl.BlockSpec((B, H_DIM), lambda i: (i, 0)),
            pl.BlockSpec((1, 1, B), lambda i: (i, 0, 0)),
        ],
        out_specs=[
            pl.BlockSpec((N_GRAPHS, H_DIM), lambda i: (0, 0)),
            pl.BlockSpec((N_GRAPHS, H_DIM), lambda i: (0, 0)),
        ],
        out_shape=[
            jax.ShapeDtypeStruct((N_GRAPHS, H_DIM), jnp.float32),
            jax.ShapeDtypeStruct((N_GRAPHS, H_DIM), jnp.float32),
        ],
    )(br, bid3)[1]


def _sc_scatter(vals, idx, n):
    N = vals.shape[0]
    K = 400
    nchunks = N // K
    assert nchunks * K == N
    NS = 16
    CD = T_DIM // 2
    ZROWS = 1000
    assert n % ZROWS == 0
    mesh = plsc.VectorSubcoreMesh(core_axis_name="c", subcore_axis_name="s")

    @functools.partial(
        pl.kernel,
        out_type=jax.ShapeDtypeStruct((1, n, T_DIM), jnp.float32),
        mesh=mesh,
        scratch_types=[
            pltpu.VMEM((K,), jnp.int32),
            pltpu.VMEM((K, CD), jnp.float32),
            pltpu.VMEM_SHARED((n, CD), jnp.float32),
        ],
        compiler_params=pltpu.CompilerParams(use_tc_tiling_on_sc=False),
    )
    def _scatter_kernel(vals_hbm, idx_hbm, zeros_hbm, out_hbm, idx_v, vals_v, accum):
        cid = lax.axis_index("c")
        sid = lax.axis_index("s")

        @pl.when(sid == 0)
        def _zero():
            def zbody(j, carry):
                pltpu.sync_copy(zeros_hbm, accum.at[pl.ds(j * ZROWS, ZROWS)])
                return carry
            lax.fori_loop(0, n // ZROWS, zbody, 0)

        plsc.subcore_barrier()
        my_chunks = nchunks // NS + jnp.where(sid < nchunks % NS, 1, 0)

        def body(i, carry):
            base = (sid + i * NS) * K
            pltpu.sync_copy(idx_hbm.at[pl.ds(base, K)], idx_v)
            pltpu.sync_copy(vals_hbm.at[pl.ds(base, K), pl.ds(cid * CD, CD)], vals_v)
            pltpu.sync_copy(vals_v, accum.at[idx_v], add=True)
            return carry

        lax.fori_loop(0, my_chunks, body, 0)
        plsc.subcore_barrier()

        @pl.when(sid == 0)
        def _writeout():
            pltpu.sync_copy(accum, out_hbm.at[0, :, pl.ds(cid * CD, CD)])

    zeros = jnp.zeros((ZROWS, CD), jnp.float32)
    return _scatter_kernel(vals, idx, zeros)


def _sc_gather(fa, fb, xp, row, col):
    G = 800
    nchunks = E // G
    NS = 16
    assert nchunks % NS == 0
    mesh = plsc.VectorSubcoreMesh(core_axis_name="c", subcore_axis_name="s")

    @functools.partial(
        pl.kernel,
        out_type=[jax.ShapeDtypeStruct((E, H_DIM), jnp.bfloat16),
                  jax.ShapeDtypeStruct((E, H_DIM), jnp.bfloat16),
                  jax.ShapeDtypeStruct((E, XD), jnp.float32),
                  jax.ShapeDtypeStruct((E, XD), jnp.float32)],
        mesh=mesh,
        scratch_types=[
            pltpu.VMEM((G,), jnp.int32),
            pltpu.VMEM((G, H_DIM), jnp.bfloat16),
            pltpu.VMEM((G, XD), jnp.float32),
            pltpu.SemaphoreType.DMA,
            pltpu.SemaphoreType.DMA,
        ],
        compiler_params=pltpu.CompilerParams(use_tc_tiling_on_sc=False),
    )
    def _gather_kernel(fa_hbm, fb_hbm, xp_hbm, row_hbm, col_hbm,
                       raf_hbm, rbf_hbm, xr_hbm, xc_hbm,
                       idx_v, fbuf, xbuf, semf, semx):
        cid = lax.axis_index("c")
        sid = lax.axis_index("s")

        def run(idx_hbm, ftab, fout, xout):
            def body(i, carry):
                base = (sid + i * NS) * G
                pltpu.sync_copy(idx_hbm.at[pl.ds(base, G)], idx_v)
                cf = pltpu.async_copy(ftab.at[idx_v], fbuf, semf)
                cx = pltpu.async_copy(xp_hbm.at[idx_v], xbuf, semx)
                cf.wait()
                cx.wait()
                pltpu.sync_copy(fbuf, fout.at[pl.ds(base, G)])
                pltpu.sync_copy(xbuf, xout.at[pl.ds(base, G)])
                return carry
            lax.fori_loop(0, nchunks // NS, body, 0)

        @pl.when(cid == 0)
        def _core0():
            run(row_hbm, fa_hbm, raf_hbm, xr_hbm)

        @pl.when(cid == 1)
        def _core1():
            run(col_hbm, fb_hbm, rbf_hbm, xc_hbm)

    return _gather_kernel(fa, fb, xp, row, col)


def kernel(H, Z, block_id, batch_id, edges, edge_attr,
           emb_in_w, emb_in_b, emb_out_w, emb_out_b,
           edge_w1, edge_b1, edge_w2, edge_b2,
           node_w1, node_b1, node_w2, node_b2,
           coord_w1, coord_b1, coord_w2):
    row, col = edges[0], edges[1]

    zp = jnp.pad(jnp.squeeze(Z, 1), ((0, 0), (0, XD - 3)))
    ones3 = (jax.lax.broadcasted_iota(jnp.int32, (1, XD), 1) == 3).astype(jnp.float32)
    atab = jnp.concatenate([H, zp + ones3], axis=1)
    s_atoms = _sc_scatter(atab, block_id, N_BLOCKS)

    wa = [edge_w1[i, :H_DIM] for i in range(3)]
    wb = [edge_w1[i, H_DIM:2 * H_DIM] for i in range(3)]
    wr = [edge_w1[i, 2 * H_DIM] for i in range(3)]
    we = [edge_w1[i, 2 * H_DIM + 1:] for i in range(3)]

    hb, xp, h, fa, fb = _k2(s_atoms, emb_in_w, emb_in_b, wa[0], wb[0], edge_b1[0])

    br = None
    for i in range(3):
        raf, rbf, xr, xc = _sc_gather(fa, fb, xp, row, col)
        out_e = _k4(raf, rbf, xr, xc, edge_attr, we[i], wr[i],
                    edge_w2[i].astype(jnp.bfloat16), edge_b2[i],
                    coord_w1[i].astype(jnp.bfloat16), coord_b1[i], coord_w2[i])
        s_e = _sc_scatter(out_e, row, N_BLOCKS)
        last = i == 2
        if last:
            nwa, nwb, nb1n = emb_out_w, emb_out_w, emb_out_b
        else:
            nwa, nwb, nb1n = wa[i + 1], wb[i + 1], edge_b1[i + 1]
        h, xp, fa, fb, br = _k6(s_e, h, xp,
                                node_w1[i, :H_DIM], node_w1[i, H_DIM:],
                                node_b1[i], node_w2[i], node_b2[i],
                                nwa, nwb, nb1n, last)

    graph_repr = _k7(br, batch_id.reshape(1, -1))
    return (hb, br, graph_repr)


# speedup vs baseline: 1.0619x; 1.0619x over previous
import functools

import jax
import jax.numpy as jnp
from jax import lax
from jax.experimental import pallas as pl
from jax.experimental.pallas import tpu as pltpu
from jax.experimental.pallas import tpu_sc as plsc

N_ATOMS = 40000
N_BLOCKS = 10000
N_GRAPHS = 256
E = 320000
H_DIM = 128
T_DIM = 144

_INTERPRET = False


def _silu(x):
    return x * jax.nn.sigmoid(x)


def _mask3(rows):
    return (jax.lax.broadcasted_iota(jnp.int32, (1, 16), 1) < 3).astype(jnp.float32)


def _onehot3(rows):
    return (jax.lax.broadcasted_iota(jnp.int32, (1, 16), 1) == 3).astype(jnp.float32)


def _k2_body(s_ref, emb_w_ref, emb_b_ref, wa_ref, wb_ref, b1_ref,
             hb_ref, xp_ref, h_ref, ta_ref, tb_ref):
    s = jnp.sum(s_ref[...], axis=0)
    cnt = jnp.maximum(s[:, 131:132], 1.0)
    hb = s[:, :H_DIM] / cnt
    xp = (s[:, H_DIM:] / cnt) * _mask3(s.shape[0])
    hb_ref[...] = hb
    xp_ref[...] = xp
    h = jnp.dot(hb, emb_w_ref[...], preferred_element_type=jnp.float32) + emb_b_ref[...]
    h_ref[...] = h
    a = jnp.dot(h, wa_ref[...], preferred_element_type=jnp.float32) + b1_ref[...]
    b = jnp.dot(h, wb_ref[...], preferred_element_type=jnp.float32)
    ta_ref[...] = jnp.concatenate([a, xp], axis=1)
    tb_ref[...] = jnp.concatenate([b, xp], axis=1)


def _k2(s, emb_w, emb_b, wa, wb, b1):
    B = 1000
    grid = (N_BLOCKS // B,)
    w_spec = pl.BlockSpec((H_DIM, H_DIM), lambda i: (0, 0))
    bias_spec = pl.BlockSpec((1, H_DIM), lambda i: (0, 0))
    return pl.pallas_call(
        _k2_body,
        grid=grid,
        in_specs=[
            pl.BlockSpec((s.shape[0], B, T_DIM), lambda i: (0, i, 0)),
            w_spec, bias_spec, w_spec, w_spec, bias_spec,
        ],
        out_specs=[
            pl.BlockSpec((B, H_DIM), lambda i: (i, 0)),
            pl.BlockSpec((B, 16), lambda i: (i, 0)),
            pl.BlockSpec((B, H_DIM), lambda i: (i, 0)),
            pl.BlockSpec((B, T_DIM), lambda i: (i, 0)),
            pl.BlockSpec((B, T_DIM), lambda i: (i, 0)),
        ],
        out_shape=[
            jax.ShapeDtypeStruct((N_BLOCKS, H_DIM), jnp.float32),
            jax.ShapeDtypeStruct((N_BLOCKS, 16), jnp.float32),
            jax.ShapeDtypeStruct((N_BLOCKS, H_DIM), jnp.float32),
            jax.ShapeDtypeStruct((N_BLOCKS, T_DIM), jnp.float32),
            jax.ShapeDtypeStruct((N_BLOCKS, T_DIM), jnp.float32),
        ],
        interpret=_INTERPRET,
    )(s, emb_w, emb_b.reshape(1, -1), wa, wb, b1.reshape(1, -1))


def _k4_body(ra_ref, rb_ref, ea_ref, we_ref, wr_ref, w2_ref, b2_ref,
             c1_ref, cb1_ref, c2_ref, out_ref):
    ra = ra_ref[...]
    rb = rb_ref[...]
    h1 = ra[:, :H_DIM] + rb[:, :H_DIM]
    d = ra[:, H_DIM:] - rb[:, H_DIM:]
    radial = jnp.sum(d * d, axis=1, keepdims=True)
    m1 = h1 + radial * wr_ref[...] + jnp.dot(
        ea_ref[...], we_ref[...], preferred_element_type=jnp.float32)
    m1 = _silu(m1).astype(jnp.bfloat16)
    m = _silu(jnp.dot(m1, w2_ref[...], preferred_element_type=jnp.float32) + b2_ref[...])
    p = _silu(jnp.dot(m.astype(jnp.bfloat16), c1_ref[...],
                      preferred_element_type=jnp.float32) + cb1_ref[...])
    phi = jnp.dot(p, c2_ref[...], preferred_element_type=jnp.float32)
    tp = d * phi + _onehot3(d.shape[0])
    out_ref[...] = jnp.concatenate([m, tp], axis=1)


def _k4(ra, rb, ea, we, wr, w2, b2, c1, cb1, c2):
    B = 4000
    grid = (E // B,)
    return pl.pallas_call(
        _k4_body,
        grid=grid,
        in_specs=[
            pl.BlockSpec((B, T_DIM), lambda i: (i, 0)),
            pl.BlockSpec((B, T_DIM), lambda i: (i, 0)),
            pl.BlockSpec((B, 16), lambda i: (i, 0)),
            pl.BlockSpec((16, H_DIM), lambda i: (0, 0)),
            pl.BlockSpec((1, H_DIM), lambda i: (0, 0)),
            pl.BlockSpec((H_DIM, H_DIM), lambda i: (0, 0)),
            pl.BlockSpec((1, H_DIM), lambda i: (0, 0)),
            pl.BlockSpec((H_DIM, H_DIM), lambda i: (0, 0)),
            pl.BlockSpec((1, H_DIM), lambda i: (0, 0)),
            pl.BlockSpec((H_DIM, 1), lambda i: (0, 0)),
        ],
        out_specs=pl.BlockSpec((B, T_DIM), lambda i: (i, 0)),
        out_shape=jax.ShapeDtypeStruct((E, T_DIM), jnp.float32),
        interpret=_INTERPRET,
    )(ra, rb, ea, we, wr.reshape(1, -1), w2, b2.reshape(1, -1),
      c1, cb1.reshape(1, -1), c2)


def _k6_body(last, s_ref, h_ref, xp_ref, wn1a_ref, wn1b_ref, nb1_ref,
             wn2_ref, nb2_ref, nwa_ref, nwb_ref, nb1n_ref,
             h_out, xp_out, ta_ref, tb_ref):
    s = jnp.sum(s_ref[...], axis=0)
    cnt = jnp.maximum(s[:, 131:132], 1.0)
    aggm = s[:, :H_DIM]
    xp = xp_ref[...] + (s[:, H_DIM:] * _mask3(s.shape[0])) / cnt
    h = h_ref[...]
    hid = _silu(jnp.dot(h, wn1a_ref[...], preferred_element_type=jnp.float32)
                + jnp.dot(aggm, wn1b_ref[...], preferred_element_type=jnp.float32)
                + nb1_ref[...])
    h = h + jnp.dot(hid, wn2_ref[...], preferred_element_type=jnp.float32) + nb2_ref[...]
    h_out[...] = h
    xp_out[...] = xp
    a = jnp.dot(h, nwa_ref[...], preferred_element_type=jnp.float32) + nb1n_ref[...]
    if last:
        nrm = jax.lax.rsqrt(jnp.maximum(jnp.sum(a * a, axis=1, keepdims=True), 1e-24))
        ta_ref[...] = jnp.concatenate([a * nrm, xp], axis=1)
        tb_ref[...] = jnp.zeros(tb_ref.shape, tb_ref.dtype)
    else:
        b = jnp.dot(h, nwb_ref[...], preferred_element_type=jnp.float32)
        ta_ref[...] = jnp.concatenate([a, xp], axis=1)
        tb_ref[...] = jnp.concatenate([b, xp], axis=1)


def _k6(s, h, xp, wn1a, wn1b, nb1, wn2, nb2, nwa, nwb, nb1n, last):
    B = 1000
    grid = (N_BLOCKS // B,)
    w_spec = pl.BlockSpec((H_DIM, H_DIM), lambda i: (0, 0))
    bias_spec = pl.BlockSpec((1, H_DIM), lambda i: (0, 0))
    return pl.pallas_call(
        functools.partial(_k6_body, last),
        grid=grid,
        in_specs=[
            pl.BlockSpec((s.shape[0], B, T_DIM), lambda i: (0, i, 0)),
            pl.BlockSpec((B, H_DIM), lambda i: (i, 0)),
            pl.BlockSpec((B, 16), lambda i: (i, 0)),
            w_spec, w_spec, bias_spec, w_spec, bias_spec,
            w_spec, w_spec, bias_spec,
        ],
        out_specs=[
            pl.BlockSpec((B, H_DIM), lambda i: (i, 0)),
            pl.BlockSpec((B, 16), lambda i: (i, 0)),
            pl.BlockSpec((B, T_DIM), lambda i: (i, 0)),
            pl.BlockSpec((B, T_DIM), lambda i: (i, 0)),
        ],
        out_shape=[
            jax.ShapeDtypeStruct((N_BLOCKS, H_DIM), jnp.float32),
            jax.ShapeDtypeStruct((N_BLOCKS, 16), jnp.float32),
            jax.ShapeDtypeStruct((N_BLOCKS, T_DIM), jnp.float32),
            jax.ShapeDtypeStruct((N_BLOCKS, T_DIM), jnp.float32),
        ],
        interpret=_INTERPRET,
    )(s, h, xp, wn1a, wn1b, nb1.reshape(1, -1), wn2, nb2.reshape(1, -1),
      nwa, nwb, nb1n.reshape(1, -1))


def _k7_body(br_ref, bid_ref, acc_ref, out_ref):
    i = pl.program_id(0)
    bid = bid_ref[0]
    onehot = (jax.lax.broadcasted_iota(jnp.int32, (N_GRAPHS, bid.shape[1]), 0)
              == bid).astype(jnp.float32)
    part = jnp.dot(onehot, br_ref[...], preferred_element_type=jnp.float32)

    @pl.when(i == 0)
    def _init():
        acc_ref[...] = part

    @pl.when(i > 0)
    def _acc():
        acc_ref[...] = acc_ref[...] + part

    @pl.when(i == pl.num_programs(0) - 1)
    def _fin():
        g = acc_ref[...]
        nrm = jax.lax.rsqrt(jnp.maximum(jnp.sum(g * g, axis=1, keepdims=True), 1e-24))
        out_ref[...] = g * nrm


def _k7(br, bid2d):
    B = 2000
    grid = (N_BLOCKS // B,)
    bid3 = bid2d.reshape(N_BLOCKS // B, 1, B)
    return pl.pallas_call(
        _k7_body,
        grid=grid,
        in_specs=[
            pl.BlockSpec((B, H_DIM), lambda i: (i, 0)),
            pl.BlockSpec((1, 1, B), lambda i: (i, 0, 0)),
        ],
        out_specs=[
            pl.BlockSpec((N_GRAPHS, H_DIM), lambda i: (0, 0)),
            pl.BlockSpec((N_GRAPHS, H_DIM), lambda i: (0, 0)),
        ],
        out_shape=[
            jax.ShapeDtypeStruct((N_GRAPHS, H_DIM), jnp.float32),
            jax.ShapeDtypeStruct((N_GRAPHS, H_DIM), jnp.float32),
        ],
        interpret=_INTERPRET,
    )(br, bid3)[1]


def _sc_scatter(vals, idx, n):
    N = vals.shape[0]
    K = 400
    nchunks = N // K
    assert nchunks * K == N
    NS = 16
    CD = T_DIM // 2
    ZROWS = 1000
    assert n % ZROWS == 0
    mesh = plsc.VectorSubcoreMesh(core_axis_name="c", subcore_axis_name="s")

    @functools.partial(
        pl.kernel,
        out_type=jax.ShapeDtypeStruct((1, n, T_DIM), jnp.float32),
        mesh=mesh,
        scratch_types=[
            pltpu.VMEM((K,), jnp.int32),
            pltpu.VMEM((K, CD), jnp.float32),
            pltpu.VMEM_SHARED((n, CD), jnp.float32),
        ],
        compiler_params=pltpu.CompilerParams(use_tc_tiling_on_sc=False),
    )
    def _scatter_kernel(vals_hbm, idx_hbm, zeros_hbm, out_hbm, idx_v, vals_v, accum):
        cid = lax.axis_index("c")
        sid = lax.axis_index("s")

        @pl.when(sid == 0)
        def _zero():
            def zbody(j, carry):
                pltpu.sync_copy(zeros_hbm, accum.at[pl.ds(j * ZROWS, ZROWS)])
                return carry
            lax.fori_loop(0, n // ZROWS, zbody, 0)

        plsc.subcore_barrier()
        my_chunks = nchunks // NS + jnp.where(sid < nchunks % NS, 1, 0)

        def body(i, carry):
            base = (sid + i * NS) * K
            pltpu.sync_copy(idx_hbm.at[pl.ds(base, K)], idx_v)
            pltpu.sync_copy(vals_hbm.at[pl.ds(base, K), pl.ds(cid * CD, CD)], vals_v)
            pltpu.sync_copy(vals_v, accum.at[idx_v], add=True)
            return carry

        lax.fori_loop(0, my_chunks, body, 0)
        plsc.subcore_barrier()

        @pl.when(sid == 0)
        def _writeout():
            pltpu.sync_copy(accum, out_hbm.at[0, :, pl.ds(cid * CD, CD)])

    zeros = jnp.zeros((ZROWS, CD), jnp.float32)
    return _scatter_kernel(vals, idx, zeros)


def _sc_gather(ta, tb, row, col):
    G = 800
    nchunks = E // G
    NS = 16
    assert nchunks % NS == 0
    mesh = plsc.VectorSubcoreMesh(core_axis_name="c", subcore_axis_name="s")

    @functools.partial(
        pl.kernel,
        out_type=[jax.ShapeDtypeStruct((E, T_DIM), jnp.float32),
                  jax.ShapeDtypeStruct((E, T_DIM), jnp.float32)],
        mesh=mesh,
        scratch_types=[
            pltpu.VMEM((G,), jnp.int32),
            pltpu.VMEM((G, T_DIM), jnp.float32),
            pltpu.SemaphoreType.DMA,
        ],
        compiler_params=pltpu.CompilerParams(use_tc_tiling_on_sc=False),
    )
    def _gather_kernel(ta_hbm, tb_hbm, row_hbm, col_hbm, ra_hbm, rb_hbm,
                       idx_v, buf, sem):
        cid = lax.axis_index("c")
        sid = lax.axis_index("s")

        def run(idx_hbm, tab, out):
            def body(i, carry):
                base = (sid + i * NS) * G
                pltpu.sync_copy(idx_hbm.at[pl.ds(base, G)], idx_v)
                pltpu.async_copy(tab.at[idx_v], buf, sem).wait()
                pltpu.sync_copy(buf, out.at[pl.ds(base, G)])
                return carry
            lax.fori_loop(0, nchunks // NS, body, 0)

        @pl.when(cid == 0)
        def _core0():
            run(row_hbm, ta_hbm, ra_hbm)

        @pl.when(cid == 1)
        def _core1():
            run(col_hbm, tb_hbm, rb_hbm)

    return _gather_kernel(ta, tb, row, col)


def _scatter_sum(vals, idx, n):
    return jax.ops.segment_sum(vals, idx, num_segments=n)[None]


def _gather(table, idx):
    return jnp.take(table, idx, axis=0)


def kernel(H, Z, block_id, batch_id, edges, edge_attr,
           emb_in_w, emb_in_b, emb_out_w, emb_out_b,
           edge_w1, edge_b1, edge_w2, edge_b2,
           node_w1, node_b1, node_w2, node_b2,
           coord_w1, coord_b1, coord_w2):
    row, col = edges[0], edges[1]

    zp = jnp.pad(jnp.squeeze(Z, 1), ((0, 0), (0, 13)))
    ones3 = (jax.lax.broadcasted_iota(jnp.int32, (1, 16), 1) == 3).astype(jnp.float32)
    atab = jnp.concatenate([H, zp + ones3], axis=1)
    s_atoms = _sc_scatter(atab, block_id, N_BLOCKS)

    wa = [edge_w1[i, :H_DIM] for i in range(3)]
    wb = [edge_w1[i, H_DIM:2 * H_DIM] for i in range(3)]
    wr = [edge_w1[i, 2 * H_DIM] for i in range(3)]
    we = [edge_w1[i, 2 * H_DIM + 1:] for i in range(3)]

    hb, xp, h, ta, tb = _k2(s_atoms, emb_in_w, emb_in_b, wa[0], wb[0], edge_b1[0])

    for i in range(3):
        ra, rb = _sc_gather(ta, tb, row, col)
        out_e = _k4(ra, rb, edge_attr, we[i], wr[i],
                    edge_w2[i].astype(jnp.bfloat16), edge_b2[i],
                    coord_w1[i].astype(jnp.bfloat16), coord_b1[i], coord_w2[i])
        s_e = _sc_scatter(out_e, row, N_BLOCKS)
        last = i == 2
        if last:
            nwa, nwb, nb1n = emb_out_w, emb_out_w, emb_out_b
        else:
            nwa, nwb, nb1n = wa[i + 1], wb[i + 1], edge_b1[i + 1]
        h, xp, ta, tb = _k6(s_e, h, xp,
                            node_w1[i, :H_DIM], node_w1[i, H_DIM:],
                            node_b1[i], node_w2[i], node_b2[i],
                            nwa, nwb, nb1n, last)

    block_repr = ta[:, :H_DIM]
    graph_repr = _k7(block_repr, batch_id.reshape(1, -1))
    return (hb, block_repr, graph_repr)


# speedup vs baseline: 1.0760x; 1.0132x over previous
import functools

import jax
import jax.numpy as jnp
from jax import lax
from jax.experimental import pallas as pl
from jax.experimental.pallas import tpu as pltpu
from jax.experimental.pallas import tpu_sc as plsc

N_ATOMS = 40000
N_BLOCKS = 10000
N_GRAPHS = 256
E = 320000
H_DIM = 128
T_DIM = 144

_INTERPRET = False


def _silu(x):
    return x * jax.nn.sigmoid(x)


def _mask3(rows):
    return (jax.lax.broadcasted_iota(jnp.int32, (1, 16), 1) < 3).astype(jnp.float32)


def _onehot3(rows):
    return (jax.lax.broadcasted_iota(jnp.int32, (1, 16), 1) == 3).astype(jnp.float32)


def _k2_body(s_ref, emb_w_ref, emb_b_ref, wa_ref, wb_ref, b1_ref,
             hb_ref, xp_ref, h_ref, ta_ref, tb_ref):
    s = jnp.sum(s_ref[...], axis=0)
    cnt = jnp.maximum(s[:, 131:132], 1.0)
    hb = s[:, :H_DIM] / cnt
    xp = (s[:, H_DIM:] / cnt) * _mask3(s.shape[0])
    hb_ref[...] = hb
    xp_ref[...] = xp
    h = jnp.dot(hb, emb_w_ref[...], preferred_element_type=jnp.float32) + emb_b_ref[...]
    h_ref[...] = h
    a = jnp.dot(h, wa_ref[...], preferred_element_type=jnp.float32) + b1_ref[...]
    b = jnp.dot(h, wb_ref[...], preferred_element_type=jnp.float32)
    ta_ref[...] = jnp.concatenate([a, xp], axis=1)
    tb_ref[...] = jnp.concatenate([b, xp], axis=1)


def _k2(s, emb_w, emb_b, wa, wb, b1):
    B = 1000
    grid = (N_BLOCKS // B,)
    w_spec = pl.BlockSpec((H_DIM, H_DIM), lambda i: (0, 0))
    bias_spec = pl.BlockSpec((1, H_DIM), lambda i: (0, 0))
    return pl.pallas_call(
        _k2_body,
        grid=grid,
        in_specs=[
            pl.BlockSpec((s.shape[0], B, T_DIM), lambda i: (0, i, 0)),
            w_spec, bias_spec, w_spec, w_spec, bias_spec,
        ],
        out_specs=[
            pl.BlockSpec((B, H_DIM), lambda i: (i, 0)),
            pl.BlockSpec((B, 16), lambda i: (i, 0)),
            pl.BlockSpec((B, H_DIM), lambda i: (i, 0)),
            pl.BlockSpec((B, T_DIM), lambda i: (i, 0)),
            pl.BlockSpec((B, T_DIM), lambda i: (i, 0)),
        ],
        out_shape=[
            jax.ShapeDtypeStruct((N_BLOCKS, H_DIM), jnp.float32),
            jax.ShapeDtypeStruct((N_BLOCKS, 16), jnp.float32),
            jax.ShapeDtypeStruct((N_BLOCKS, H_DIM), jnp.float32),
            jax.ShapeDtypeStruct((N_BLOCKS, T_DIM), jnp.float32),
            jax.ShapeDtypeStruct((N_BLOCKS, T_DIM), jnp.float32),
        ],
        interpret=_INTERPRET,
    )(s, emb_w, emb_b.reshape(1, -1), wa, wb, b1.reshape(1, -1))


def _k4_body(ra_ref, rb_ref, ea_ref, we_ref, wr_ref, w2_ref, b2_ref,
             c1_ref, cb1_ref, c2_ref, out_ref):
    ra = ra_ref[...]
    rb = rb_ref[...]
    h1 = ra[:, :H_DIM] + rb[:, :H_DIM]
    d = ra[:, H_DIM:] - rb[:, H_DIM:]
    radial = jnp.sum(d * d, axis=1, keepdims=True)
    m1 = h1 + radial * wr_ref[...] + jnp.dot(
        ea_ref[...], we_ref[...], preferred_element_type=jnp.float32)
    m1 = _silu(m1).astype(jnp.bfloat16)
    m = _silu(jnp.dot(m1, w2_ref[...], preferred_element_type=jnp.float32) + b2_ref[...])
    p = _silu(jnp.dot(m.astype(jnp.bfloat16), c1_ref[...],
                      preferred_element_type=jnp.float32) + cb1_ref[...])
    phi = jnp.dot(p, c2_ref[...], preferred_element_type=jnp.float32)
    tp = d * phi + _onehot3(d.shape[0])
    out_ref[...] = jnp.concatenate([m, tp], axis=1)


def _k4(ra, rb, ea, we, wr, w2, b2, c1, cb1, c2):
    B = 4000
    grid = (E // B,)
    return pl.pallas_call(
        _k4_body,
        grid=grid,
        in_specs=[
            pl.BlockSpec((B, T_DIM), lambda i: (i, 0)),
            pl.BlockSpec((B, T_DIM), lambda i: (i, 0)),
            pl.BlockSpec((B, 16), lambda i: (i, 0)),
            pl.BlockSpec((16, H_DIM), lambda i: (0, 0)),
            pl.BlockSpec((1, H_DIM), lambda i: (0, 0)),
            pl.BlockSpec((H_DIM, H_DIM), lambda i: (0, 0)),
            pl.BlockSpec((1, H_DIM), lambda i: (0, 0)),
            pl.BlockSpec((H_DIM, H_DIM), lambda i: (0, 0)),
            pl.BlockSpec((1, H_DIM), lambda i: (0, 0)),
            pl.BlockSpec((H_DIM, 1), lambda i: (0, 0)),
        ],
        out_specs=pl.BlockSpec((B, T_DIM), lambda i: (i, 0)),
        out_shape=jax.ShapeDtypeStruct((E, T_DIM), jnp.float32),
        interpret=_INTERPRET,
    )(ra, rb, ea, we, wr.reshape(1, -1), w2, b2.reshape(1, -1),
      c1, cb1.reshape(1, -1), c2)


def _k6_body(last, s_ref, h_ref, xp_ref, wn1a_ref, wn1b_ref, nb1_ref,
             wn2_ref, nb2_ref, nwa_ref, nwb_ref, nb1n_ref,
             h_out, xp_out, ta_ref, tb_ref):
    s = jnp.sum(s_ref[...], axis=0)
    cnt = jnp.maximum(s[:, 131:132], 1.0)
    aggm = s[:, :H_DIM]
    xp = xp_ref[...] + (s[:, H_DIM:] * _mask3(s.shape[0])) / cnt
    h = h_ref[...]
    hid = _silu(jnp.dot(h, wn1a_ref[...], preferred_element_type=jnp.float32)
                + jnp.dot(aggm, wn1b_ref[...], preferred_element_type=jnp.float32)
                + nb1_ref[...])
    h = h + jnp.dot(hid, wn2_ref[...], preferred_element_type=jnp.float32) + nb2_ref[...]
    h_out[...] = h
    xp_out[...] = xp
    a = jnp.dot(h, nwa_ref[...], preferred_element_type=jnp.float32) + nb1n_ref[...]
    if last:
        nrm = jax.lax.rsqrt(jnp.maximum(jnp.sum(a * a, axis=1, keepdims=True), 1e-24))
        ta_ref[...] = jnp.concatenate([a * nrm, xp], axis=1)
        tb_ref[...] = jnp.zeros(tb_ref.shape, tb_ref.dtype)
    else:
        b = jnp.dot(h, nwb_ref[...], preferred_element_type=jnp.float32)
        ta_ref[...] = jnp.concatenate([a, xp], axis=1)
        tb_ref[...] = jnp.concatenate([b, xp], axis=1)


def _k6(s, h, xp, wn1a, wn1b, nb1, wn2, nb2, nwa, nwb, nb1n, last):
    B = 1000
    grid = (N_BLOCKS // B,)
    w_spec = pl.BlockSpec((H_DIM, H_DIM), lambda i: (0, 0))
    bias_spec = pl.BlockSpec((1, H_DIM), lambda i: (0, 0))
    return pl.pallas_call(
        functools.partial(_k6_body, last),
        grid=grid,
        in_specs=[
            pl.BlockSpec((s.shape[0], B, T_DIM), lambda i: (0, i, 0)),
            pl.BlockSpec((B, H_DIM), lambda i: (i, 0)),
            pl.BlockSpec((B, 16), lambda i: (i, 0)),
            w_spec, w_spec, bias_spec, w_spec, bias_spec,
            w_spec, w_spec, bias_spec,
        ],
        out_specs=[
            pl.BlockSpec((B, H_DIM), lambda i: (i, 0)),
            pl.BlockSpec((B, 16), lambda i: (i, 0)),
            pl.BlockSpec((B, T_DIM), lambda i: (i, 0)),
            pl.BlockSpec((B, T_DIM), lambda i: (i, 0)),
        ],
        out_shape=[
            jax.ShapeDtypeStruct((N_BLOCKS, H_DIM), jnp.float32),
            jax.ShapeDtypeStruct((N_BLOCKS, 16), jnp.float32),
            jax.ShapeDtypeStruct((N_BLOCKS, T_DIM), jnp.float32),
            jax.ShapeDtypeStruct((N_BLOCKS, T_DIM), jnp.float32),
        ],
        interpret=_INTERPRET,
    )(s, h, xp, wn1a, wn1b, nb1.reshape(1, -1), wn2, nb2.reshape(1, -1),
      nwa, nwb, nb1n.reshape(1, -1))


def _k7_body(br_ref, bid_ref, acc_ref, out_ref):
    i = pl.program_id(0)
    bid = bid_ref[0]
    onehot = (jax.lax.broadcasted_iota(jnp.int32, (N_GRAPHS, bid.shape[1]), 0)
              == bid).astype(jnp.float32)
    part = jnp.dot(onehot, br_ref[...], preferred_element_type=jnp.float32)

    @pl.when(i == 0)
    def _init():
        acc_ref[...] = part

    @pl.when(i > 0)
    def _acc():
        acc_ref[...] = acc_ref[...] + part

    @pl.when(i == pl.num_programs(0) - 1)
    def _fin():
        g = acc_ref[...]
        nrm = jax.lax.rsqrt(jnp.maximum(jnp.sum(g * g, axis=1, keepdims=True), 1e-24))
        out_ref[...] = g * nrm


def _k7(br, bid2d):
    B = 2000
    grid = (N_BLOCKS // B,)
    bid3 = bid2d.reshape(N_BLOCKS // B, 1, B)
    return pl.pallas_call(
        _k7_body,
        grid=grid,
        in_specs=[
            pl.BlockSpec((B, H_DIM), lambda i: (i, 0)),
            pl.BlockSpec((1, 1, B), lambda i: (i, 0, 0)),
        ],
        out_specs=[
            pl.BlockSpec((N_GRAPHS, H_DIM), lambda i: (0, 0)),
            pl.BlockSpec((N_GRAPHS, H_DIM), lambda i: (0, 0)),
        ],
        out_shape=[
            jax.ShapeDtypeStruct((N_GRAPHS, H_DIM), jnp.float32),
            jax.ShapeDtypeStruct((N_GRAPHS, H_DIM), jnp.float32),
        ],
        interpret=_INTERPRET,
    )(br, bid3)[1]


def _sc_scatter(vals, idx, n):
    N = vals.shape[0]
    K = 800
    nchunks = N // K
    assert nchunks * K == N
    NS = 16
    CD = T_DIM // 2
    ZROWS = 1000
    assert n % ZROWS == 0
    mesh = plsc.VectorSubcoreMesh(core_axis_name="c", subcore_axis_name="s")

    @functools.partial(
        pl.kernel,
        out_type=jax.ShapeDtypeStruct((1, n, T_DIM), jnp.float32),
        mesh=mesh,
        scratch_types=[
            pltpu.VMEM((K,), jnp.int32),
            pltpu.VMEM((K, CD), jnp.float32),
            pltpu.VMEM_SHARED((n, CD), jnp.float32),
        ],
        compiler_params=pltpu.CompilerParams(use_tc_tiling_on_sc=False),
    )
    def _scatter_kernel(vals_hbm, idx_hbm, zeros_hbm, out_hbm, idx_v, vals_v, accum):
        cid = lax.axis_index("c")
        sid = lax.axis_index("s")

        @pl.when(sid == 0)
        def _zero():
            def zbody(j, carry):
                pltpu.sync_copy(zeros_hbm, accum.at[pl.ds(j * ZROWS, ZROWS)])
                return carry
            lax.fori_loop(0, n // ZROWS, zbody, 0)

        plsc.subcore_barrier()
        my_chunks = nchunks // NS + jnp.where(sid < nchunks % NS, 1, 0)

        def body(i, carry):
            base = (sid + i * NS) * K
            pltpu.sync_copy(idx_hbm.at[pl.ds(base, K)], idx_v)
            pltpu.sync_copy(vals_hbm.at[pl.ds(base, K), pl.ds(cid * CD, CD)], vals_v)
            pltpu.sync_copy(vals_v, accum.at[idx_v], add=True)
            return carry

        lax.fori_loop(0, my_chunks, body, 0)
        plsc.subcore_barrier()

        @pl.when(sid == 0)
        def _writeout():
            pltpu.sync_copy(accum, out_hbm.at[0, :, pl.ds(cid * CD, CD)])

    zeros = jnp.zeros((ZROWS, CD), jnp.float32)
    return _scatter_kernel(vals, idx, zeros)


def _sc_gather(ta, tb, row, col):
    G = 800
    nchunks = E // G
    NS = 16
    assert nchunks % NS == 0
    mesh = plsc.VectorSubcoreMesh(core_axis_name="c", subcore_axis_name="s")

    @functools.partial(
        pl.kernel,
        out_type=[jax.ShapeDtypeStruct((E, T_DIM), jnp.float32),
                  jax.ShapeDtypeStruct((E, T_DIM), jnp.float32)],
        mesh=mesh,
        scratch_types=[
            pltpu.VMEM((G,), jnp.int32),
            pltpu.VMEM((G, T_DIM), jnp.float32),
            pltpu.SemaphoreType.DMA,
        ],
        compiler_params=pltpu.CompilerParams(use_tc_tiling_on_sc=False),
    )
    def _gather_kernel(ta_hbm, tb_hbm, row_hbm, col_hbm, ra_hbm, rb_hbm,
                       idx_v, buf, sem):
        cid = lax.axis_index("c")
        sid = lax.axis_index("s")

        def run(idx_hbm, tab, out):
            def body(i, carry):
                base = (sid + i * NS) * G
                pltpu.sync_copy(idx_hbm.at[pl.ds(base, G)], idx_v)
                pltpu.async_copy(tab.at[idx_v], buf, sem).wait()
                pltpu.sync_copy(buf, out.at[pl.ds(base, G)])
                return carry
            lax.fori_loop(0, nchunks // NS, body, 0)

        @pl.when(cid == 0)
        def _core0():
            run(row_hbm, ta_hbm, ra_hbm)

        @pl.when(cid == 1)
        def _core1():
            run(col_hbm, tb_hbm, rb_hbm)

    return _gather_kernel(ta, tb, row, col)


def _scatter_sum(vals, idx, n):
    return jax.ops.segment_sum(vals, idx, num_segments=n)[None]


def _gather(table, idx):
    return jnp.take(table, idx, axis=0)


def kernel(H, Z, block_id, batch_id, edges, edge_attr,
           emb_in_w, emb_in_b, emb_out_w, emb_out_b,
           edge_w1, edge_b1, edge_w2, edge_b2,
           node_w1, node_b1, node_w2, node_b2,
           coord_w1, coord_b1, coord_w2):
    row, col = edges[0], edges[1]

    zp = jnp.pad(jnp.squeeze(Z, 1), ((0, 0), (0, 13)))
    ones3 = (jax.lax.broadcasted_iota(jnp.int32, (1, 16), 1) == 3).astype(jnp.float32)
    atab = jnp.concatenate([H, zp + ones3], axis=1)
    s_atoms = _sc_scatter(atab, block_id, N_BLOCKS)

    wa = [edge_w1[i, :H_DIM] for i in range(3)]
    wb = [edge_w1[i, H_DIM:2 * H_DIM] for i in range(3)]
    wr = [edge_w1[i, 2 * H_DIM] for i in range(3)]
    we = [edge_w1[i, 2 * H_DIM + 1:] for i in range(3)]

    hb, xp, h, ta, tb = _k2(s_atoms, emb_in_w, emb_in_b, wa[0], wb[0], edge_b1[0])

    for i in range(3):
        ra, rb = _sc_gather(ta, tb, row, col)
        out_e = _k4(ra, rb, edge_attr, we[i], wr[i],
                    edge_w2[i].astype(jnp.bfloat16), edge_b2[i],
                    coord_w1[i].astype(jnp.bfloat16), coord_b1[i], coord_w2[i])
        s_e = _sc_scatter(out_e, row, N_BLOCKS)
        last = i == 2
        if last:
            nwa, nwb, nb1n = emb_out_w, emb_out_w, emb_out_b
        else:
            nwa, nwb, nb1n = wa[i + 1], wb[i + 1], edge_b1[i + 1]
        h, xp, ta, tb = _k6(s_e, h, xp,
                            node_w1[i, :H_DIM], node_w1[i, H_DIM:],
                            node_b1[i], node_w2[i], node_b2[i],
                            nwa, nwb, nb1n, last)

    block_repr = ta[:, :H_DIM]
    graph_repr = _k7(block_repr, batch_id.reshape(1, -1))
    return (hb, block_repr, graph_repr)


# speedup vs baseline: 1.0762x; 1.0003x over previous
import functools

import jax
import jax.numpy as jnp
from jax import lax
from jax.experimental import pallas as pl
from jax.experimental.pallas import tpu as pltpu
from jax.experimental.pallas import tpu_sc as plsc

N_ATOMS = 40000
N_BLOCKS = 10000
N_GRAPHS = 256
E = 320000
H_DIM = 128
T_DIM = 144


def _silu(x):
    return x * jax.nn.sigmoid(x)


def _mask3(rows):
    return (jax.lax.broadcasted_iota(jnp.int32, (1, 16), 1) < 3).astype(jnp.float32)


def _onehot3(rows):
    return (jax.lax.broadcasted_iota(jnp.int32, (1, 16), 1) == 3).astype(jnp.float32)


def _k2_body(s_ref, emb_w_ref, emb_b_ref, wa_ref, wb_ref, b1_ref,
             hb_ref, xp_ref, h_ref, ta_ref, tb_ref):
    s = jnp.sum(s_ref[...], axis=0)
    cnt = jnp.maximum(s[:, 131:132], 1.0)
    hb = s[:, :H_DIM] / cnt
    xp = (s[:, H_DIM:] / cnt) * _mask3(s.shape[0])
    hb_ref[...] = hb
    xp_ref[...] = xp
    h = jnp.dot(hb, emb_w_ref[...], preferred_element_type=jnp.float32) + emb_b_ref[...]
    h_ref[...] = h
    a = jnp.dot(h, wa_ref[...], preferred_element_type=jnp.float32) + b1_ref[...]
    b = jnp.dot(h, wb_ref[...], preferred_element_type=jnp.float32)
    ta_ref[...] = jnp.concatenate([a, xp], axis=1)
    tb_ref[...] = jnp.concatenate([b, xp], axis=1)


def _k2(s, emb_w, emb_b, wa, wb, b1):
    B = 1000
    grid = (N_BLOCKS // B,)
    w_spec = pl.BlockSpec((H_DIM, H_DIM), lambda i: (0, 0))
    bias_spec = pl.BlockSpec((1, H_DIM), lambda i: (0, 0))
    return pl.pallas_call(
        _k2_body,
        grid=grid,
        in_specs=[
            pl.BlockSpec((s.shape[0], B, T_DIM), lambda i: (0, i, 0)),
            w_spec, bias_spec, w_spec, w_spec, bias_spec,
        ],
        out_specs=[
            pl.BlockSpec((B, H_DIM), lambda i: (i, 0)),
            pl.BlockSpec((B, 16), lambda i: (i, 0)),
            pl.BlockSpec((B, H_DIM), lambda i: (i, 0)),
            pl.BlockSpec((B, T_DIM), lambda i: (i, 0)),
            pl.BlockSpec((B, T_DIM), lambda i: (i, 0)),
        ],
        out_shape=[
            jax.ShapeDtypeStruct((N_BLOCKS, H_DIM), jnp.float32),
            jax.ShapeDtypeStruct((N_BLOCKS, 16), jnp.float32),
            jax.ShapeDtypeStruct((N_BLOCKS, H_DIM), jnp.float32),
            jax.ShapeDtypeStruct((N_BLOCKS, T_DIM), jnp.float32),
            jax.ShapeDtypeStruct((N_BLOCKS, T_DIM), jnp.float32),
        ],
    )(s, emb_w, emb_b.reshape(1, -1), wa, wb, b1.reshape(1, -1))


def _k4_body(ra_ref, rb_ref, ea_ref, we_ref, wr_ref, w2_ref, b2_ref,
             c1_ref, cb1_ref, c2_ref, out_ref):
    ra = ra_ref[...]
    rb = rb_ref[...]
    h1 = ra[:, :H_DIM] + rb[:, :H_DIM]
    d = ra[:, H_DIM:] - rb[:, H_DIM:]
    radial = jnp.sum(d * d, axis=1, keepdims=True)
    m1 = h1 + radial * wr_ref[...] + jnp.dot(
        ea_ref[...], we_ref[...], preferred_element_type=jnp.float32)
    m1 = _silu(m1).astype(jnp.bfloat16)
    m = _silu(jnp.dot(m1, w2_ref[...], preferred_element_type=jnp.float32) + b2_ref[...])
    p = _silu(jnp.dot(m.astype(jnp.bfloat16), c1_ref[...],
                      preferred_element_type=jnp.float32) + cb1_ref[...])
    phi = jnp.dot(p, c2_ref[...], preferred_element_type=jnp.float32)
    tp = d * phi + _onehot3(d.shape[0])
    out_ref[...] = jnp.concatenate([m, tp], axis=1)


def _k4(ra, rb, ea, we, wr, w2, b2, c1, cb1, c2):
    B = 4000
    grid = (E // B,)
    return pl.pallas_call(
        _k4_body,
        grid=grid,
        in_specs=[
            pl.BlockSpec((B, T_DIM), lambda i: (i, 0)),
            pl.BlockSpec((B, T_DIM), lambda i: (i, 0)),
            pl.BlockSpec((B, 16), lambda i: (i, 0)),
            pl.BlockSpec((16, H_DIM), lambda i: (0, 0)),
            pl.BlockSpec((1, H_DIM), lambda i: (0, 0)),
            pl.BlockSpec((H_DIM, H_DIM), lambda i: (0, 0)),
            pl.BlockSpec((1, H_DIM), lambda i: (0, 0)),
            pl.BlockSpec((H_DIM, H_DIM), lambda i: (0, 0)),
            pl.BlockSpec((1, H_DIM), lambda i: (0, 0)),
            pl.BlockSpec((H_DIM, 1), lambda i: (0, 0)),
        ],
        out_specs=pl.BlockSpec((B, T_DIM), lambda i: (i, 0)),
        out_shape=jax.ShapeDtypeStruct((E, T_DIM), jnp.float32),
    )(ra, rb, ea, we, wr.reshape(1, -1), w2, b2.reshape(1, -1),
      c1, cb1.reshape(1, -1), c2)


def _k6_body(last, s_ref, h_ref, xp_ref, wn1a_ref, wn1b_ref, nb1_ref,
             wn2_ref, nb2_ref, nwa_ref, nwb_ref, nb1n_ref,
             h_out, xp_out, ta_ref, tb_ref):
    s = jnp.sum(s_ref[...], axis=0)
    cnt = jnp.maximum(s[:, 131:132], 1.0)
    aggm = s[:, :H_DIM]
    xp = xp_ref[...] + (s[:, H_DIM:] * _mask3(s.shape[0])) / cnt
    h = h_ref[...]
    hid = _silu(jnp.dot(h, wn1a_ref[...], preferred_element_type=jnp.float32)
                + jnp.dot(aggm, wn1b_ref[...], preferred_element_type=jnp.float32)
                + nb1_ref[...])
    h = h + jnp.dot(hid, wn2_ref[...], preferred_element_type=jnp.float32) + nb2_ref[...]
    h_out[...] = h
    xp_out[...] = xp
    a = jnp.dot(h, nwa_ref[...], preferred_element_type=jnp.float32) + nb1n_ref[...]
    if last:
        nrm = jax.lax.rsqrt(jnp.maximum(jnp.sum(a * a, axis=1, keepdims=True), 1e-24))
        ta_ref[...] = jnp.concatenate([a * nrm, xp], axis=1)
        tb_ref[...] = jnp.zeros(tb_ref.shape, tb_ref.dtype)
    else:
        b = jnp.dot(h, nwb_ref[...], preferred_element_type=jnp.float32)
        ta_ref[...] = jnp.concatenate([a, xp], axis=1)
        tb_ref[...] = jnp.concatenate([b, xp], axis=1)


def _k6(s, h, xp, wn1a, wn1b, nb1, wn2, nb2, nwa, nwb, nb1n, last):
    B = 1000
    grid = (N_BLOCKS // B,)
    w_spec = pl.BlockSpec((H_DIM, H_DIM), lambda i: (0, 0))
    bias_spec = pl.BlockSpec((1, H_DIM), lambda i: (0, 0))
    return pl.pallas_call(
        functools.partial(_k6_body, last),
        grid=grid,
        in_specs=[
            pl.BlockSpec((s.shape[0], B, T_DIM), lambda i: (0, i, 0)),
            pl.BlockSpec((B, H_DIM), lambda i: (i, 0)),
            pl.BlockSpec((B, 16), lambda i: (i, 0)),
            w_spec, w_spec, bias_spec, w_spec, bias_spec,
            w_spec, w_spec, bias_spec,
        ],
        out_specs=[
            pl.BlockSpec((B, H_DIM), lambda i: (i, 0)),
            pl.BlockSpec((B, 16), lambda i: (i, 0)),
            pl.BlockSpec((B, T_DIM), lambda i: (i, 0)),
            pl.BlockSpec((B, T_DIM), lambda i: (i, 0)),
        ],
        out_shape=[
            jax.ShapeDtypeStruct((N_BLOCKS, H_DIM), jnp.float32),
            jax.ShapeDtypeStruct((N_BLOCKS, 16), jnp.float32),
            jax.ShapeDtypeStruct((N_BLOCKS, T_DIM), jnp.float32),
            jax.ShapeDtypeStruct((N_BLOCKS, T_DIM), jnp.float32),
        ],
    )(s, h, xp, wn1a, wn1b, nb1.reshape(1, -1), wn2, nb2.reshape(1, -1),
      nwa, nwb, nb1n.reshape(1, -1))


def _k7_body(br_ref, bid_ref, acc_ref, out_ref):
    i = pl.program_id(0)
    bid = bid_ref[0]
    onehot = (jax.lax.broadcasted_iota(jnp.int32, (N_GRAPHS, bid.shape[1]), 0)
              == bid).astype(jnp.float32)
    part = jnp.dot(onehot, br_ref[...], preferred_element_type=jnp.float32)

    @pl.when(i == 0)
    def _init():
        acc_ref[...] = part

    @pl.when(i > 0)
    def _acc():
        acc_ref[...] = acc_ref[...] + part

    @pl.when(i == pl.num_programs(0) - 1)
    def _fin():
        g = acc_ref[...]
        nrm = jax.lax.rsqrt(jnp.maximum(jnp.sum(g * g, axis=1, keepdims=True), 1e-24))
        out_ref[...] = g * nrm


def _k7(br, bid2d):
    B = 2000
    grid = (N_BLOCKS // B,)
    bid3 = bid2d.reshape(N_BLOCKS // B, 1, B)
    return pl.pallas_call(
        _k7_body,
        grid=grid,
        in_specs=[
            pl.BlockSpec((B, H_DIM), lambda i: (i, 0)),
            pl.BlockSpec((1, 1, B), lambda i: (i, 0, 0)),
        ],
        out_specs=[
            pl.BlockSpec((N_GRAPHS, H_DIM), lambda i: (0, 0)),
            pl.BlockSpec((N_GRAPHS, H_DIM), lambda i: (0, 0)),
        ],
        out_shape=[
            jax.ShapeDtypeStruct((N_GRAPHS, H_DIM), jnp.float32),
            jax.ShapeDtypeStruct((N_GRAPHS, H_DIM), jnp.float32),
        ],
    )(br, bid3)[1]


def _sc_scatter(vals, idx, n):
    N = vals.shape[0]
    K = 800
    nchunks = N // K
    assert nchunks * K == N
    NS = 16
    CD = T_DIM // 2
    ZROWS = 1000
    assert n % ZROWS == 0
    mesh = plsc.VectorSubcoreMesh(core_axis_name="c", subcore_axis_name="s")

    @functools.partial(
        pl.kernel,
        out_type=jax.ShapeDtypeStruct((1, n, T_DIM), jnp.float32),
        mesh=mesh,
        scratch_types=[
            pltpu.VMEM((K,), jnp.int32),
            pltpu.VMEM((K, CD), jnp.float32),
            pltpu.VMEM_SHARED((n, CD), jnp.float32),
        ],
        compiler_params=pltpu.CompilerParams(use_tc_tiling_on_sc=False),
    )
    def _scatter_kernel(vals_hbm, idx_hbm, zeros_hbm, out_hbm, idx_v, vals_v, accum):
        cid = lax.axis_index("c")
        sid = lax.axis_index("s")

        @pl.when(sid == 0)
        def _zero():
            def zbody(j, carry):
                pltpu.sync_copy(zeros_hbm, accum.at[pl.ds(j * ZROWS, ZROWS)])
                return carry
            lax.fori_loop(0, n // ZROWS, zbody, 0)

        plsc.subcore_barrier()
        my_chunks = nchunks // NS + jnp.where(sid < nchunks % NS, 1, 0)

        def body(i, carry):
            base = (sid + i * NS) * K
            pltpu.sync_copy(idx_hbm.at[pl.ds(base, K)], idx_v)
            pltpu.sync_copy(vals_hbm.at[pl.ds(base, K), pl.ds(cid * CD, CD)], vals_v)
            pltpu.sync_copy(vals_v, accum.at[idx_v], add=True)
            return carry

        lax.fori_loop(0, my_chunks, body, 0)
        plsc.subcore_barrier()

        @pl.when(sid == 0)
        def _writeout():
            pltpu.sync_copy(accum, out_hbm.at[0, :, pl.ds(cid * CD, CD)])

    zeros = jnp.zeros((ZROWS, CD), jnp.float32)
    return _scatter_kernel(vals, idx, zeros)


def _sc_gather(ta, tb, row, col):
    G = 800
    nchunks = E // G
    NS = 16
    assert nchunks % NS == 0
    mesh = plsc.VectorSubcoreMesh(core_axis_name="c", subcore_axis_name="s")

    @functools.partial(
        pl.kernel,
        out_type=[jax.ShapeDtypeStruct((E, T_DIM), jnp.float32),
                  jax.ShapeDtypeStruct((E, T_DIM), jnp.float32)],
        mesh=mesh,
        scratch_types=[
            pltpu.VMEM((G,), jnp.int32),
            pltpu.VMEM((G, T_DIM), jnp.float32),
            pltpu.SemaphoreType.DMA,
        ],
        compiler_params=pltpu.CompilerParams(use_tc_tiling_on_sc=False),
    )
    def _gather_kernel(ta_hbm, tb_hbm, row_hbm, col_hbm, ra_hbm, rb_hbm,
                       idx_v, buf, sem):
        cid = lax.axis_index("c")
        sid = lax.axis_index("s")

        def run(idx_hbm, tab, out):
            def body(i, carry):
                base = (sid + i * NS) * G
                pltpu.sync_copy(idx_hbm.at[pl.ds(base, G)], idx_v)
                pltpu.async_copy(tab.at[idx_v], buf, sem).wait()
                pltpu.sync_copy(buf, out.at[pl.ds(base, G)])
                return carry
            lax.fori_loop(0, nchunks // NS, body, 0)

        @pl.when(cid == 0)
        def _core0():
            run(row_hbm, ta_hbm, ra_hbm)

        @pl.when(cid == 1)
        def _core1():
            run(col_hbm, tb_hbm, rb_hbm)

    return _gather_kernel(ta, tb, row, col)


def kernel(H, Z, block_id, batch_id, edges, edge_attr,
           emb_in_w, emb_in_b, emb_out_w, emb_out_b,
           edge_w1, edge_b1, edge_w2, edge_b2,
           node_w1, node_b1, node_w2, node_b2,
           coord_w1, coord_b1, coord_w2):
    row, col = edges[0], edges[1]

    zp = jnp.pad(jnp.squeeze(Z, 1), ((0, 0), (0, 13)))
    ones3 = (jax.lax.broadcasted_iota(jnp.int32, (1, 16), 1) == 3).astype(jnp.float32)
    atab = jnp.concatenate([H, zp + ones3], axis=1)
    s_atoms = _sc_scatter(atab, block_id, N_BLOCKS)

    wa = [edge_w1[i, :H_DIM] for i in range(3)]
    wb = [edge_w1[i, H_DIM:2 * H_DIM] for i in range(3)]
    wr = [edge_w1[i, 2 * H_DIM] for i in range(3)]
    we = [edge_w1[i, 2 * H_DIM + 1:] for i in range(3)]

    hb, xp, h, ta, tb = _k2(s_atoms, emb_in_w, emb_in_b, wa[0], wb[0], edge_b1[0])

    for i in range(3):
        ra, rb = _sc_gather(ta, tb, row, col)
        out_e = _k4(ra, rb, edge_attr, we[i], wr[i],
                    edge_w2[i].astype(jnp.bfloat16), edge_b2[i],
                    coord_w1[i].astype(jnp.bfloat16), coord_b1[i], coord_w2[i])
        s_e = _sc_scatter(out_e, row, N_BLOCKS)
        last = i == 2
        if last:
            nwa, nwb, nb1n = emb_out_w, emb_out_w, emb_out_b
        else:
            nwa, nwb, nb1n = wa[i + 1], wb[i + 1], edge_b1[i + 1]
        h, xp, ta, tb = _k6(s_e, h, xp,
                            node_w1[i, :H_DIM], node_w1[i, H_DIM:],
                            node_b1[i], node_w2[i], node_b2[i],
                            nwa, nwb, nb1n, last)

    block_repr = ta[:, :H_DIM]
    graph_repr = _k7(block_repr, batch_id.reshape(1, -1))
    return (hb, block_repr, graph_repr)
